# Initial kernel scaffold; baseline (speedup 1.0000x reference)
import jax
import jax.numpy as jnp
from jax.experimental import pallas as pl


def kernel(xyz, new_xyz):
    raise NotImplementedError("write your pallas kernel here")


# speedup vs baseline: 1.1484x; 1.1484x over previous
import functools

import jax
import jax.numpy as jnp
import numpy as np
from jax.experimental import pallas as pl

RADIUS = 0.2
NSAMPLE = 32
QB = 128


def _cross(ax, ay, az, bx, by, bz):
    return (ay * bz - az * by, az * bx - ax * bz, ax * by - ay * bx)


def _bitonic_sort_tiles(vals):
    n = len(vals)
    k = 2
    while k <= n:
        j = k // 2
        while j >= 1:
            for i in range(n):
                l = i ^ j
                if l > i:
                    up = (i & k) == 0
                    a, b = vals[i], vals[l]
                    lo = jnp.minimum(a, b)
                    hi = jnp.maximum(a, b)
                    vals[i], vals[l] = (lo, hi) if up else (hi, lo)
            j //= 2
        k *= 2
    return vals


def _rri_kernel(xr, yr, zr, outr):
    px = xr[0, :32, :]
    py = yr[0, :32, :]
    pz = zr[0, :32, :]
    cx = xr[0, 32:33, :]
    cy = yr[0, 32:33, :]
    cz = zr[0, 32:33, :]

    pxi = px[:, None, :]
    pyi = py[:, None, :]
    pzi = pz[:, None, :]
    dx = pxi - px[None, :, :]
    dy = pyi - py[None, :, :]
    dz = pzi - pz[None, :, :]
    dis = jnp.sqrt((dx * dx + dy * dy) + dz * dz)

    mean_dis = jnp.sum(dis, axis=1) * jnp.float32(1.0 / 32.0)
    mx = jnp.max(mean_dis, axis=0, keepdims=True)
    ii = jax.lax.broadcasted_iota(jnp.int32, (32, QB), 0)
    tip = jnp.min(jnp.where(mean_dis == mx, ii, jnp.int32(32)), axis=0,
                  keepdims=True)
    sel = ii == tip
    zero = jnp.zeros((32, QB), jnp.float32)
    tx = jnp.sum(jnp.where(sel, px, zero), axis=0, keepdims=True)
    ty = jnp.sum(jnp.where(sel, py, zero), axis=0, keepdims=True)
    tz = jnp.sum(jnp.where(sel, pz, zero), axis=0, keepdims=True)

    ux, uy, uz = _cross(cx, cy, cz, px, py, pz)
    gx, gy, gz = _cross(ux, uy, uz, cx, cy, cz)
    gn = jnp.sqrt((gx * gx + gy * gy) + gz * gz)
    gx, gy, gz = gx / gn, gy / gn, gz / gn

    vx, vy, vz = _cross(cx, cy, cz, tx, ty, tz)
    tpx, tpy, tpz = _cross(vx, vy, vz, cx, cy, cz)
    tn = jnp.sqrt((tpx * tpx + tpy * tpy) + tpz * tpz)
    tpx, tpy, tpz = tpx / tn, tpy / tn, tpz / tn

    wx, wy, wz = _cross(gx, gy, gz, tpx, tpy, tpz)
    cr = jnp.sqrt((cx * cx + cy * cy) + cz * cz)
    inv = 1.0 / (cr + jnp.float32(1e-8))
    nnx, nny, nnz = cx * inv, cy * inv, cz * inv
    gp_sin = (wx * nnx + wy * nny) + wz * nnz

    tiles = [dis[i] for i in range(32)]
    tiles = _bitonic_sort_tiles(tiles)
    rows = [t * gp_sin for t in tiles]

    rows.append(jnp.sqrt((px * px + py * py) + pz * pz))
    outr[0] = jnp.stack(rows, axis=0)


def _rri_features(gxT, gyT, gzT, B, NP):
    out = pl.pallas_call(
        _rri_kernel,
        out_shape=jax.ShapeDtypeStruct((B, 33, 32, NP), jnp.float32),
        grid=(B, NP // QB),
        in_specs=[
            pl.BlockSpec((1, 33, QB), lambda b, q: (b, 0, q)),
            pl.BlockSpec((1, 33, QB), lambda b, q: (b, 0, q)),
            pl.BlockSpec((1, 33, QB), lambda b, q: (b, 0, q)),
        ],
        out_specs=pl.BlockSpec((1, 33, 32, QB), lambda b, q: (b, 0, 0, q)),
    )(gxT, gyT, gzT)
    return jnp.transpose(out, (0, 1, 3, 2))


def _ball_query_jax(xyz, new_xyz):
    B, N, _ = xyz.shape
    xx = jnp.sum(xyz * xyz, axis=-1)
    nn = jnp.sum(new_xyz * new_xyz, axis=-1)
    d2 = nn[:, :, None] + xx[:, None, :] - 2.0 * jnp.einsum(
        'bpc,bnc->bpn', new_xyz, xyz)
    mask = d2 < (RADIUS * RADIUS)
    ar = jnp.arange(N, dtype=jnp.int32)
    score = jnp.where(mask, -ar[None, None, :], jnp.int32(-N))
    vals, _ = jax.lax.top_k(score, NSAMPLE)
    idx_sorted = -vals
    first = idx_sorted[:, :, :1]
    idx = jnp.where(idx_sorted == N, first, idx_sorted)
    idx = jnp.where(idx == N, 0, idx)
    return idx


def kernel(xyz, new_xyz):
    B, N, _ = xyz.shape
    NP = new_xyz.shape[1]
    idx = _ball_query_jax(xyz, new_xyz)
    grouped = jax.vmap(lambda f, i: f[i])(xyz, idx)
    g = jnp.transpose(grouped, (0, 3, 2, 1))
    q = jnp.transpose(new_xyz, (0, 2, 1))[:, :, None, :]
    packed = jnp.concatenate([g, q], axis=2)
    return _rri_features(packed[:, 0], packed[:, 1], packed[:, 2], B, NP)


# speedup vs baseline: 59.9015x; 52.1592x over previous
import functools

import jax
import jax.numpy as jnp
import numpy as np
from jax import lax
from jax.experimental import pallas as pl
from jax.experimental.pallas import tpu as pltpu
from jax.experimental.pallas import tpu_sc as plsc

RADIUS = 0.2
NSAMPLE = 32
QB = 128


def _cross(ax, ay, az, bx, by, bz):
    return (ay * bz - az * by, az * bx - ax * bz, ax * by - ay * bx)


def _bitonic_sort_tiles(vals):
    n = len(vals)
    k = 2
    while k <= n:
        j = k // 2
        while j >= 1:
            for i in range(n):
                l = i ^ j
                if l > i:
                    up = (i & k) == 0
                    a, b = vals[i], vals[l]
                    lo = jnp.minimum(a, b)
                    hi = jnp.maximum(a, b)
                    vals[i], vals[l] = (lo, hi) if up else (hi, lo)
            j //= 2
        k *= 2
    return vals


def _rri_kernel(xr, yr, zr, outr):
    px = xr[0, :32, :]
    py = yr[0, :32, :]
    pz = zr[0, :32, :]
    cx = xr[0, 32:33, :]
    cy = yr[0, 32:33, :]
    cz = zr[0, 32:33, :]

    pxi = px[:, None, :]
    pyi = py[:, None, :]
    pzi = pz[:, None, :]
    dx = pxi - px[None, :, :]
    dy = pyi - py[None, :, :]
    dz = pzi - pz[None, :, :]
    dis = jnp.sqrt((dx * dx + dy * dy) + dz * dz)

    mean_dis = jnp.sum(dis, axis=1) * jnp.float32(1.0 / 32.0)
    mx = jnp.max(mean_dis, axis=0, keepdims=True)
    ii = jax.lax.broadcasted_iota(jnp.int32, (32, QB), 0)
    tip = jnp.min(jnp.where(mean_dis == mx, ii, jnp.int32(32)), axis=0,
                  keepdims=True)
    sel = ii == tip
    zero = jnp.zeros((32, QB), jnp.float32)
    tx = jnp.sum(jnp.where(sel, px, zero), axis=0, keepdims=True)
    ty = jnp.sum(jnp.where(sel, py, zero), axis=0, keepdims=True)
    tz = jnp.sum(jnp.where(sel, pz, zero), axis=0, keepdims=True)

    ux, uy, uz = _cross(cx, cy, cz, px, py, pz)
    gx, gy, gz = _cross(ux, uy, uz, cx, cy, cz)
    gn = jnp.sqrt((gx * gx + gy * gy) + gz * gz)
    gx, gy, gz = gx / gn, gy / gn, gz / gn

    vx, vy, vz = _cross(cx, cy, cz, tx, ty, tz)
    tpx, tpy, tpz = _cross(vx, vy, vz, cx, cy, cz)
    tn = jnp.sqrt((tpx * tpx + tpy * tpy) + tpz * tpz)
    tpx, tpy, tpz = tpx / tn, tpy / tn, tpz / tn

    wx, wy, wz = _cross(gx, gy, gz, tpx, tpy, tpz)
    cr = jnp.sqrt((cx * cx + cy * cy) + cz * cz)
    inv = 1.0 / (cr + jnp.float32(1e-8))
    nnx, nny, nnz = cx * inv, cy * inv, cz * inv
    gp_sin = (wx * nnx + wy * nny) + wz * nnz

    tiles = [dis[i] for i in range(32)]
    tiles = _bitonic_sort_tiles(tiles)
    rows = [t * gp_sin for t in tiles]

    rows.append(jnp.sqrt((px * px + py * py) + pz * pz))
    outr[0] = jnp.stack(rows, axis=0)


def _rri_features(gxT, gyT, gzT, B, NP):
    out = pl.pallas_call(
        _rri_kernel,
        out_shape=jax.ShapeDtypeStruct((B, 33, 32, NP), jnp.float32),
        grid=(B, NP // QB),
        in_specs=[
            pl.BlockSpec((1, 33, QB), lambda b, q: (b, 0, q)),
            pl.BlockSpec((1, 33, QB), lambda b, q: (b, 0, q)),
            pl.BlockSpec((1, 33, QB), lambda b, q: (b, 0, q)),
        ],
        out_specs=pl.BlockSpec((1, 33, 32, QB), lambda b, q: (b, 0, 0, q)),
    )(gxT, gyT, gzT)
    return jnp.transpose(out, (0, 1, 3, 2))


def _ball_group_sc(xs, ys, zs, xx, qx, qy, qz, nn, B, N, NP):
    info = plsc.get_sparse_core_info()
    NC, NS = info.num_cores, info.num_subcores
    NW = NC * NS
    TQ = (B * NP) // NW
    TPB = NW // B
    r2 = np.float32(RADIUS * RADIUS)
    nchunks = N // 16

    mesh = plsc.VectorSubcoreMesh(core_axis_name="c", subcore_axis_name="s")

    @functools.partial(
        pl.kernel,
        mesh=mesh,
        compiler_params=pltpu.CompilerParams(needs_layout_passes=False),
        out_type=[jax.ShapeDtypeStruct((B * NP * 32,), jnp.float32)] * 3,
        scratch_types=[
            pltpu.VMEM((N,), jnp.float32),
            pltpu.VMEM((N,), jnp.float32),
            pltpu.VMEM((N,), jnp.float32),
            pltpu.VMEM((N,), jnp.float32),
            pltpu.VMEM((TQ + 16,), jnp.float32),
            pltpu.VMEM((TQ + 16,), jnp.float32),
            pltpu.VMEM((TQ + 16,), jnp.float32),
            pltpu.VMEM((TQ + 16,), jnp.float32),
            pltpu.VMEM((48,), jnp.int32),
            pltpu.VMEM((TQ * 32,), jnp.float32),
            pltpu.VMEM((TQ * 32,), jnp.float32),
            pltpu.VMEM((TQ * 32,), jnp.float32),
        ],
    )
    def k(xs_h, ys_h, zs_h, xx_h, qx_h, qy_h, qz_h, nn_h,
          gx_h, gy_h, gz_h,
          xs_v, ys_v, zs_v, xx_v, qx_v, qy_v, qz_v, nn_v, buf, ox, oy, oz):
        wid = lax.axis_index("s") * NC + lax.axis_index("c")
        qbase = wid * TQ
        pbase = (wid // TPB) * N
        pltpu.sync_copy(xs_h.at[pl.ds(pbase, N)], xs_v)
        pltpu.sync_copy(ys_h.at[pl.ds(pbase, N)], ys_v)
        pltpu.sync_copy(zs_h.at[pl.ds(pbase, N)], zs_v)
        pltpu.sync_copy(xx_h.at[pl.ds(pbase, N)], xx_v)
        pltpu.sync_copy(qx_h.at[pl.ds(qbase, TQ)], qx_v.at[pl.ds(0, TQ)])
        pltpu.sync_copy(qy_h.at[pl.ds(qbase, TQ)], qy_v.at[pl.ds(0, TQ)])
        pltpu.sync_copy(qz_h.at[pl.ds(qbase, TQ)], qz_v.at[pl.ds(0, TQ)])
        pltpu.sync_copy(nn_h.at[pl.ds(qbase, TQ)], nn_v.at[pl.ds(0, TQ)])

        lane = lax.iota(jnp.int32, 16)

        def bf16r(v):
            t = lax.bitcast_convert_type(v, jnp.int32)
            r = (t + jnp.int32(0x7FFF) + ((t >> 16) & 1)) & jnp.int32(-65536)
            return lax.bitcast_convert_type(r, jnp.float32)

        def per_query(q, carry):
            qxs = qx_v[pl.ds(q, 16)][0]
            qys = qy_v[pl.ds(q, 16)][0]
            qzs = qz_v[pl.ds(q, 16)][0]
            nns = nn_v[pl.ds(q, 16)][0]

            def cond(st):
                i, off = st
                return (off < 32) & (i < nchunks)

            def body(st):
                i, off = st
                base = i * 16
                px = bf16r(xs_v[pl.ds(base, 16)])
                py = bf16r(ys_v[pl.ds(base, 16)])
                pz = bf16r(zs_v[pl.ds(base, 16)])
                xxv = xx_v[pl.ds(base, 16)]
                dot = (qxs * px + qys * py) + qzs * pz
                d2 = (nns + xxv) - jnp.float32(2.0) * dot
                m = d2 < r2
                ids = lane + base
                plsc.store_compressed(buf.at[pl.ds(off, 16)], ids, mask=m)
                c = jnp.sum(m.astype(jnp.int32))
                return i + 1, off + c

            _, off = lax.while_loop(cond, body, (jnp.int32(0), jnp.int32(0)))

            first = jnp.where(off > 0, buf[pl.ds(0, 16)][0], jnp.int32(0))
            i0 = buf[pl.ds(0, 16)]
            i1 = buf[pl.ds(16, 16)]
            i0 = jnp.where(lane < off, i0, first)
            i1 = jnp.where(lane + 16 < off, i1, first)
            qo = q * 32
            ox[pl.ds(qo, 16)] = plsc.load_gather(xs_v, [i0])
            ox[pl.ds(qo + 16, 16)] = plsc.load_gather(xs_v, [i1])
            oy[pl.ds(qo, 16)] = plsc.load_gather(ys_v, [i0])
            oy[pl.ds(qo + 16, 16)] = plsc.load_gather(ys_v, [i1])
            oz[pl.ds(qo, 16)] = plsc.load_gather(zs_v, [i0])
            oz[pl.ds(qo + 16, 16)] = plsc.load_gather(zs_v, [i1])
            return carry

        lax.fori_loop(0, TQ, per_query, 0)
        pltpu.sync_copy(ox, gx_h.at[pl.ds(qbase * 32, TQ * 32)])
        pltpu.sync_copy(oy, gy_h.at[pl.ds(qbase * 32, TQ * 32)])
        pltpu.sync_copy(oz, gz_h.at[pl.ds(qbase * 32, TQ * 32)])

    return k(xs, ys, zs, xx, qx, qy, qz, nn)


def kernel(xyz, new_xyz):
    B, N, _ = xyz.shape
    NP = new_xyz.shape[1]
    xt = jnp.transpose(xyz, (0, 2, 1))
    qt = jnp.transpose(new_xyz, (0, 2, 1))
    xx = jnp.sum(xyz * xyz, axis=-1)
    nn = jnp.sum(new_xyz * new_xyz, axis=-1)
    qtb = lax.bitcast_convert_type(qt, jnp.int32)
    qtb = (qtb + jnp.int32(0x7FFF) + ((qtb >> 16) & 1)) & jnp.int32(-65536)
    qb = lax.bitcast_convert_type(qtb, jnp.float32)
    gx, gy, gz = _ball_group_sc(
        xt[:, 0].reshape(-1), xt[:, 1].reshape(-1), xt[:, 2].reshape(-1),
        xx.reshape(-1),
        qb[:, 0].reshape(-1), qb[:, 1].reshape(-1), qb[:, 2].reshape(-1),
        nn.reshape(-1), B, N, NP)
    X = jnp.concatenate(
        [jnp.transpose(gx.reshape(B, NP, 32), (0, 2, 1)), qt[:, 0:1]], axis=1)
    Y = jnp.concatenate(
        [jnp.transpose(gy.reshape(B, NP, 32), (0, 2, 1)), qt[:, 1:2]], axis=1)
    Z = jnp.concatenate(
        [jnp.transpose(gz.reshape(B, NP, 32), (0, 2, 1)), qt[:, 2:3]], axis=1)
    return _rri_features(X, Y, Z, B, NP)


# speedup vs baseline: 136.0150x; 2.2706x over previous
import functools

import jax
import jax.numpy as jnp
import numpy as np
from jax import lax
from jax.experimental import pallas as pl
from jax.experimental.pallas import tpu as pltpu
from jax.experimental.pallas import tpu_sc as plsc

RADIUS = 0.2
NSAMPLE = 32
QB = 128


def _cross(ax, ay, az, bx, by, bz):
    return (ay * bz - az * by, az * bx - ax * bz, ax * by - ay * bx)


def _bitonic_sort_tiles(vals):
    n = len(vals)
    k = 2
    while k <= n:
        j = k // 2
        while j >= 1:
            for i in range(n):
                l = i ^ j
                if l > i:
                    up = (i & k) == 0
                    a, b = vals[i], vals[l]
                    lo = jnp.minimum(a, b)
                    hi = jnp.maximum(a, b)
                    vals[i], vals[l] = (lo, hi) if up else (hi, lo)
            j //= 2
        k *= 2
    return vals


def _rri_kernel(xr, yr, zr, outr):
    px = xr[0, :32, :]
    py = yr[0, :32, :]
    pz = zr[0, :32, :]
    cx = xr[0, 32:33, :]
    cy = yr[0, 32:33, :]
    cz = zr[0, 32:33, :]

    pxi = px[:, None, :]
    pyi = py[:, None, :]
    pzi = pz[:, None, :]
    dx = pxi - px[None, :, :]
    dy = pyi - py[None, :, :]
    dz = pzi - pz[None, :, :]
    dis = jnp.sqrt((dx * dx + dy * dy) + dz * dz)

    mean_dis = jnp.sum(dis, axis=1) * jnp.float32(1.0 / 32.0)
    mx = jnp.max(mean_dis, axis=0, keepdims=True)
    ii = jax.lax.broadcasted_iota(jnp.int32, (32, QB), 0)
    tip = jnp.min(jnp.where(mean_dis == mx, ii, jnp.int32(32)), axis=0,
                  keepdims=True)
    sel = ii == tip
    zero = jnp.zeros((32, QB), jnp.float32)
    tx = jnp.sum(jnp.where(sel, px, zero), axis=0, keepdims=True)
    ty = jnp.sum(jnp.where(sel, py, zero), axis=0, keepdims=True)
    tz = jnp.sum(jnp.where(sel, pz, zero), axis=0, keepdims=True)

    ux, uy, uz = _cross(cx, cy, cz, px, py, pz)
    gx, gy, gz = _cross(ux, uy, uz, cx, cy, cz)
    gn = jnp.sqrt((gx * gx + gy * gy) + gz * gz)
    gx, gy, gz = gx / gn, gy / gn, gz / gn

    vx, vy, vz = _cross(cx, cy, cz, tx, ty, tz)
    tpx, tpy, tpz = _cross(vx, vy, vz, cx, cy, cz)
    tn = jnp.sqrt((tpx * tpx + tpy * tpy) + tpz * tpz)
    tpx, tpy, tpz = tpx / tn, tpy / tn, tpz / tn

    wx, wy, wz = _cross(gx, gy, gz, tpx, tpy, tpz)
    cr = jnp.sqrt((cx * cx + cy * cy) + cz * cz)
    inv = 1.0 / (cr + jnp.float32(1e-8))
    nnx, nny, nnz = cx * inv, cy * inv, cz * inv
    gp_sin = (wx * nnx + wy * nny) + wz * nnz

    tiles = [dis[i] for i in range(32)]
    tiles = _bitonic_sort_tiles(tiles)
    rows = [t * gp_sin for t in tiles]

    rows.append(jnp.sqrt((px * px + py * py) + pz * pz))
    outr[0] = jnp.stack(rows, axis=0)


def _rri_features(gxT, gyT, gzT, B, NP):
    out = pl.pallas_call(
        _rri_kernel,
        out_shape=jax.ShapeDtypeStruct((B, 33, 32, NP), jnp.float32),
        grid=(B, NP // QB),
        in_specs=[
            pl.BlockSpec((1, 33, QB), lambda b, q: (b, 0, q)),
            pl.BlockSpec((1, 33, QB), lambda b, q: (b, 0, q)),
            pl.BlockSpec((1, 33, QB), lambda b, q: (b, 0, q)),
        ],
        out_specs=pl.BlockSpec((1, 33, 32, QB), lambda b, q: (b, 0, 0, q)),
    )(gxT, gyT, gzT)
    return jnp.transpose(out, (0, 1, 3, 2))


def _ball_group_sc(xs, ys, zs, xx, qx, qy, qz, nn, B, N, NP):
    info = plsc.get_sparse_core_info()
    NC, NS = info.num_cores, info.num_subcores
    NW = NC * NS
    TQ = (B * NP) // NW
    TPB = NW // B
    r2 = np.float32(RADIUS * RADIUS)
    nchunks = N // 16

    mesh = plsc.VectorSubcoreMesh(core_axis_name="c", subcore_axis_name="s")

    @functools.partial(
        pl.kernel,
        mesh=mesh,
        compiler_params=pltpu.CompilerParams(needs_layout_passes=False),
        out_type=[jax.ShapeDtypeStruct((B * NP * 32,), jnp.float32)] * 3,
        scratch_types=[
            pltpu.VMEM((N,), jnp.float32),
            pltpu.VMEM((N,), jnp.float32),
            pltpu.VMEM((N,), jnp.float32),
            pltpu.VMEM((N,), jnp.float32),
            pltpu.VMEM((TQ + 16,), jnp.float32),
            pltpu.VMEM((TQ + 16,), jnp.float32),
            pltpu.VMEM((TQ + 16,), jnp.float32),
            pltpu.VMEM((TQ + 16,), jnp.float32),
            pltpu.VMEM((112,), jnp.int32),
            pltpu.VMEM((TQ * 32,), jnp.float32),
            pltpu.VMEM((TQ * 32,), jnp.float32),
            pltpu.VMEM((TQ * 32,), jnp.float32),
        ],
    )
    def k(xs_h, ys_h, zs_h, xx_h, qx_h, qy_h, qz_h, nn_h,
          gx_h, gy_h, gz_h,
          xs_v, ys_v, zs_v, xx_v, qx_v, qy_v, qz_v, nn_v, buf, ox, oy, oz):
        wid = lax.axis_index("s") * NC + lax.axis_index("c")
        qbase = wid * TQ
        pbase = (wid // TPB) * N
        pltpu.sync_copy(xs_h.at[pl.ds(pbase, N)], xs_v)
        pltpu.sync_copy(ys_h.at[pl.ds(pbase, N)], ys_v)
        pltpu.sync_copy(zs_h.at[pl.ds(pbase, N)], zs_v)
        pltpu.sync_copy(xx_h.at[pl.ds(pbase, N)], xx_v)
        pltpu.sync_copy(qx_h.at[pl.ds(qbase, TQ)], qx_v.at[pl.ds(0, TQ)])
        pltpu.sync_copy(qy_h.at[pl.ds(qbase, TQ)], qy_v.at[pl.ds(0, TQ)])
        pltpu.sync_copy(qz_h.at[pl.ds(qbase, TQ)], qz_v.at[pl.ds(0, TQ)])
        pltpu.sync_copy(nn_h.at[pl.ds(qbase, TQ)], nn_v.at[pl.ds(0, TQ)])

        lane = lax.iota(jnp.int32, 16)

        def bf16r(v):
            t = lax.bitcast_convert_type(v, jnp.int32)
            r = (t + jnp.int32(0x7FFF) + ((t >> 16) & 1)) & jnp.int32(-65536)
            return lax.bitcast_convert_type(r, jnp.float32)

        def per_query(q, carry):
            qxs = qx_v[pl.ds(q, 16)][0]
            qys = qy_v[pl.ds(q, 16)][0]
            qzs = qz_v[pl.ds(q, 16)][0]
            nns = nn_v[pl.ds(q, 16)][0]

            def cond(st):
                i, off = st
                return (off < 32) & (i < nchunks // 4)

            def body(st):
                i, off = st
                base = i * 64
                parts = []
                for u in range(4):
                    sub = base + u * 16
                    px = bf16r(xs_v[pl.ds(sub, 16)])
                    py = bf16r(ys_v[pl.ds(sub, 16)])
                    pz = bf16r(zs_v[pl.ds(sub, 16)])
                    xxv = xx_v[pl.ds(sub, 16)]
                    dot = (qxs * px + qys * py) + qzs * pz
                    d2 = (nns + xxv) - jnp.float32(2.0) * dot
                    m = d2 < r2
                    c = plsc.all_reduce_population_count(m)[0]
                    parts.append((m, lane + sub, c))
                o = off
                for m, ids, c in parts:
                    plsc.store_compressed(buf.at[pl.ds(o, 16)], ids, mask=m)
                    o = o + c
                return i + 1, o

            _, off = lax.while_loop(cond, body, (jnp.int32(0), jnp.int32(0)))

            first = jnp.where(off > 0, buf[pl.ds(0, 16)][0], jnp.int32(0))
            i0 = buf[pl.ds(0, 16)]
            i1 = buf[pl.ds(16, 16)]
            i0 = jnp.where(lane < off, i0, first)
            i1 = jnp.where(lane + 16 < off, i1, first)
            qo = q * 32
            ox[pl.ds(qo, 16)] = plsc.load_gather(xs_v, [i0])
            ox[pl.ds(qo + 16, 16)] = plsc.load_gather(xs_v, [i1])
            oy[pl.ds(qo, 16)] = plsc.load_gather(ys_v, [i0])
            oy[pl.ds(qo + 16, 16)] = plsc.load_gather(ys_v, [i1])
            oz[pl.ds(qo, 16)] = plsc.load_gather(zs_v, [i0])
            oz[pl.ds(qo + 16, 16)] = plsc.load_gather(zs_v, [i1])
            return carry

        lax.fori_loop(0, TQ, per_query, 0)
        pltpu.sync_copy(ox, gx_h.at[pl.ds(qbase * 32, TQ * 32)])
        pltpu.sync_copy(oy, gy_h.at[pl.ds(qbase * 32, TQ * 32)])
        pltpu.sync_copy(oz, gz_h.at[pl.ds(qbase * 32, TQ * 32)])

    return k(xs, ys, zs, xx, qx, qy, qz, nn)


def kernel(xyz, new_xyz):
    B, N, _ = xyz.shape
    NP = new_xyz.shape[1]
    xt = jnp.transpose(xyz, (0, 2, 1))
    qt = jnp.transpose(new_xyz, (0, 2, 1))
    xx = jnp.sum(xyz * xyz, axis=-1)
    nn = jnp.sum(new_xyz * new_xyz, axis=-1)
    qtb = lax.bitcast_convert_type(qt, jnp.int32)
    qtb = (qtb + jnp.int32(0x7FFF) + ((qtb >> 16) & 1)) & jnp.int32(-65536)
    qb = lax.bitcast_convert_type(qtb, jnp.float32)
    gx, gy, gz = _ball_group_sc(
        xt[:, 0].reshape(-1), xt[:, 1].reshape(-1), xt[:, 2].reshape(-1),
        xx.reshape(-1),
        qb[:, 0].reshape(-1), qb[:, 1].reshape(-1), qb[:, 2].reshape(-1),
        nn.reshape(-1), B, N, NP)
    X = jnp.concatenate(
        [jnp.transpose(gx.reshape(B, NP, 32), (0, 2, 1)), qt[:, 0:1]], axis=1)
    Y = jnp.concatenate(
        [jnp.transpose(gy.reshape(B, NP, 32), (0, 2, 1)), qt[:, 1:2]], axis=1)
    Z = jnp.concatenate(
        [jnp.transpose(gz.reshape(B, NP, 32), (0, 2, 1)), qt[:, 2:3]], axis=1)
    return _rri_features(X, Y, Z, B, NP)


# speedup vs baseline: 140.2776x; 1.0313x over previous
import functools

import jax
import jax.numpy as jnp
import numpy as np
from jax import lax
from jax.experimental import pallas as pl
from jax.experimental.pallas import tpu as pltpu
from jax.experimental.pallas import tpu_sc as plsc

RADIUS = 0.2
NSAMPLE = 32
QB = 128


def _cross(ax, ay, az, bx, by, bz):
    return (ay * bz - az * by, az * bx - ax * bz, ax * by - ay * bx)


def _bitonic_sort_tiles(vals):
    n = len(vals)
    k = 2
    while k <= n:
        j = k // 2
        while j >= 1:
            for i in range(n):
                l = i ^ j
                if l > i:
                    up = (i & k) == 0
                    a, b = vals[i], vals[l]
                    lo = jnp.minimum(a, b)
                    hi = jnp.maximum(a, b)
                    vals[i], vals[l] = (lo, hi) if up else (hi, lo)
            j //= 2
        k *= 2
    return vals


def _rri_kernel(xr, yr, zr, outr):
    px = xr[0, :32, :]
    py = yr[0, :32, :]
    pz = zr[0, :32, :]
    cx = xr[0, 32:33, :]
    cy = yr[0, 32:33, :]
    cz = zr[0, 32:33, :]

    pxi = px[:, None, :]
    pyi = py[:, None, :]
    pzi = pz[:, None, :]
    dx = pxi - px[None, :, :]
    dy = pyi - py[None, :, :]
    dz = pzi - pz[None, :, :]
    dis = jnp.sqrt((dx * dx + dy * dy) + dz * dz)

    mean_dis = jnp.sum(dis, axis=1) * jnp.float32(1.0 / 32.0)
    mx = jnp.max(mean_dis, axis=0, keepdims=True)
    ii = jax.lax.broadcasted_iota(jnp.int32, (32, QB), 0)
    tip = jnp.min(jnp.where(mean_dis == mx, ii, jnp.int32(32)), axis=0,
                  keepdims=True)
    sel = ii == tip
    zero = jnp.zeros((32, QB), jnp.float32)
    tx = jnp.sum(jnp.where(sel, px, zero), axis=0, keepdims=True)
    ty = jnp.sum(jnp.where(sel, py, zero), axis=0, keepdims=True)
    tz = jnp.sum(jnp.where(sel, pz, zero), axis=0, keepdims=True)

    ux, uy, uz = _cross(cx, cy, cz, px, py, pz)
    gx, gy, gz = _cross(ux, uy, uz, cx, cy, cz)
    gn = jnp.sqrt((gx * gx + gy * gy) + gz * gz)
    gx, gy, gz = gx / gn, gy / gn, gz / gn

    vx, vy, vz = _cross(cx, cy, cz, tx, ty, tz)
    tpx, tpy, tpz = _cross(vx, vy, vz, cx, cy, cz)
    tn = jnp.sqrt((tpx * tpx + tpy * tpy) + tpz * tpz)
    tpx, tpy, tpz = tpx / tn, tpy / tn, tpz / tn

    wx, wy, wz = _cross(gx, gy, gz, tpx, tpy, tpz)
    cr = jnp.sqrt((cx * cx + cy * cy) + cz * cz)
    inv = 1.0 / (cr + jnp.float32(1e-8))
    nnx, nny, nnz = cx * inv, cy * inv, cz * inv
    gp_sin = (wx * nnx + wy * nny) + wz * nnz

    tiles = [dis[i] for i in range(32)]
    tiles = _bitonic_sort_tiles(tiles)
    rows = [t * gp_sin for t in tiles]

    rows.append(jnp.sqrt((px * px + py * py) + pz * pz))
    outr[0] = jnp.stack(rows, axis=0)


def _rri_features(gxT, gyT, gzT, B, NP):
    out = pl.pallas_call(
        _rri_kernel,
        out_shape=jax.ShapeDtypeStruct((B, 33, 32, NP), jnp.float32),
        grid=(B, NP // QB),
        in_specs=[
            pl.BlockSpec((1, 33, QB), lambda b, q: (b, 0, q)),
            pl.BlockSpec((1, 33, QB), lambda b, q: (b, 0, q)),
            pl.BlockSpec((1, 33, QB), lambda b, q: (b, 0, q)),
        ],
        out_specs=pl.BlockSpec((1, 33, 32, QB), lambda b, q: (b, 0, 0, q)),
    )(gxT, gyT, gzT)
    return jnp.transpose(out, (0, 1, 3, 2))


def _ball_group_sc(xs, ys, zs, xx, qx, qy, qz, nn, B, N, NP):
    info = plsc.get_sparse_core_info()
    NC, NS = info.num_cores, info.num_subcores
    NW = NC * NS
    TQ = (B * NP) // NW
    TPB = NW // B
    r2 = np.float32(RADIUS * RADIUS)
    nchunks = N // 16

    mesh = plsc.VectorSubcoreMesh(core_axis_name="c", subcore_axis_name="s")

    @functools.partial(
        pl.kernel,
        mesh=mesh,
        compiler_params=pltpu.CompilerParams(needs_layout_passes=False),
        out_type=[jax.ShapeDtypeStruct((B * NP * 32,), jnp.float32)] * 3,
        scratch_types=[
            pltpu.VMEM((N,), jnp.float32),
            pltpu.VMEM((N,), jnp.float32),
            pltpu.VMEM((N,), jnp.float32),
            pltpu.VMEM((N,), jnp.float32),
            pltpu.VMEM((TQ + 16,), jnp.float32),
            pltpu.VMEM((TQ + 16,), jnp.float32),
            pltpu.VMEM((TQ + 16,), jnp.float32),
            pltpu.VMEM((TQ + 16,), jnp.float32),
            pltpu.VMEM((8 * 64,), jnp.int32),
            pltpu.VMEM((TQ * 32,), jnp.float32),
            pltpu.VMEM((TQ * 32,), jnp.float32),
            pltpu.VMEM((TQ * 32,), jnp.float32),
        ],
    )
    def k(xs_h, ys_h, zs_h, xx_h, qx_h, qy_h, qz_h, nn_h,
          gx_h, gy_h, gz_h,
          xs_v, ys_v, zs_v, xx_v, qx_v, qy_v, qz_v, nn_v, buf, ox, oy, oz):
        wid = lax.axis_index("s") * NC + lax.axis_index("c")
        qbase = wid * TQ
        pbase = (wid // TPB) * N
        pltpu.sync_copy(xs_h.at[pl.ds(pbase, N)], xs_v)
        pltpu.sync_copy(ys_h.at[pl.ds(pbase, N)], ys_v)
        pltpu.sync_copy(zs_h.at[pl.ds(pbase, N)], zs_v)
        pltpu.sync_copy(xx_h.at[pl.ds(pbase, N)], xx_v)
        pltpu.sync_copy(qx_h.at[pl.ds(qbase, TQ)], qx_v.at[pl.ds(0, TQ)])
        pltpu.sync_copy(qy_h.at[pl.ds(qbase, TQ)], qy_v.at[pl.ds(0, TQ)])
        pltpu.sync_copy(qz_h.at[pl.ds(qbase, TQ)], qz_v.at[pl.ds(0, TQ)])
        pltpu.sync_copy(nn_h.at[pl.ds(qbase, TQ)], nn_v.at[pl.ds(0, TQ)])

        lane = lax.iota(jnp.int32, 16)

        def bf16r(v):
            t = lax.bitcast_convert_type(v, jnp.int32)
            r = (t + jnp.int32(0x7FFF) + ((t >> 16) & 1)) & jnp.int32(-65536)
            return lax.bitcast_convert_type(r, jnp.float32)

        QG = 8

        def per_block(blk, carry):
            qb0 = blk * QG
            qxv = qx_v[pl.ds(qb0, 16)]
            qyv = qy_v[pl.ds(qb0, 16)]
            qzv = qz_v[pl.ds(qb0, 16)]
            nnv = nn_v[pl.ds(qb0, 16)]
            qxs = [qxv[j] for j in range(QG)]
            qys = [qyv[j] for j in range(QG)]
            qzs = [qzv[j] for j in range(QG)]
            nns = [nnv[j] for j in range(QG)]

            def cond(st):
                i = st[0]
                offs = st[1:]
                active = offs[0] < 32
                for o in offs[1:]:
                    active = active | (o < 32)
                return active & (i < nchunks)

            def body(st):
                i = st[0]
                offs = list(st[1:])
                base = i * 16
                px = bf16r(xs_v[pl.ds(base, 16)])
                py = bf16r(ys_v[pl.ds(base, 16)])
                pz = bf16r(zs_v[pl.ds(base, 16)])
                xxv = xx_v[pl.ds(base, 16)]
                ids = lane + base
                for j in range(QG):
                    dot = (qxs[j] * px + qys[j] * py) + qzs[j] * pz
                    d2 = (nns[j] + xxv) - jnp.float32(2.0) * dot
                    m = d2 < r2
                    c = plsc.all_reduce_population_count(m)[0]
                    o = offs[j]
                    plsc.store_compressed(buf.at[pl.ds(j * 64 + o, 16)],
                                          ids, mask=m)
                    offs[j] = jnp.where(o < 32, o + c, o)
                return (i + 1, *offs)

            st = lax.while_loop(cond, body, (jnp.int32(0),) + (jnp.int32(0),) * QG)
            offs = st[1:]

            for j in range(QG):
                off = offs[j]
                jb = j * 64
                i0 = buf[pl.ds(jb, 16)]
                i1 = buf[pl.ds(jb + 16, 16)]
                first = jnp.where(off > 0, i0[0], jnp.int32(0))
                i0 = jnp.where(lane < off, i0, first)
                i1 = jnp.where(lane + 16 < off, i1, first)
                qo = (qb0 + j) * 32
                ox[pl.ds(qo, 16)] = plsc.load_gather(xs_v, [i0])
                ox[pl.ds(qo + 16, 16)] = plsc.load_gather(xs_v, [i1])
                oy[pl.ds(qo, 16)] = plsc.load_gather(ys_v, [i0])
                oy[pl.ds(qo + 16, 16)] = plsc.load_gather(ys_v, [i1])
                oz[pl.ds(qo, 16)] = plsc.load_gather(zs_v, [i0])
                oz[pl.ds(qo + 16, 16)] = plsc.load_gather(zs_v, [i1])
            return carry

        lax.fori_loop(0, TQ // QG, per_block, 0)
        pltpu.sync_copy(ox, gx_h.at[pl.ds(qbase * 32, TQ * 32)])
        pltpu.sync_copy(oy, gy_h.at[pl.ds(qbase * 32, TQ * 32)])
        pltpu.sync_copy(oz, gz_h.at[pl.ds(qbase * 32, TQ * 32)])

    return k(xs, ys, zs, xx, qx, qy, qz, nn)


def kernel(xyz, new_xyz):
    B, N, _ = xyz.shape
    NP = new_xyz.shape[1]
    xt = jnp.transpose(xyz, (0, 2, 1))
    qt = jnp.transpose(new_xyz, (0, 2, 1))
    xx = jnp.sum(xyz * xyz, axis=-1)
    nn = jnp.sum(new_xyz * new_xyz, axis=-1)
    qtb = lax.bitcast_convert_type(qt, jnp.int32)
    qtb = (qtb + jnp.int32(0x7FFF) + ((qtb >> 16) & 1)) & jnp.int32(-65536)
    qb = lax.bitcast_convert_type(qtb, jnp.float32)
    gx, gy, gz = _ball_group_sc(
        xt[:, 0].reshape(-1), xt[:, 1].reshape(-1), xt[:, 2].reshape(-1),
        xx.reshape(-1),
        qb[:, 0].reshape(-1), qb[:, 1].reshape(-1), qb[:, 2].reshape(-1),
        nn.reshape(-1), B, N, NP)
    X = jnp.concatenate(
        [jnp.transpose(gx.reshape(B, NP, 32), (0, 2, 1)), qt[:, 0:1]], axis=1)
    Y = jnp.concatenate(
        [jnp.transpose(gy.reshape(B, NP, 32), (0, 2, 1)), qt[:, 1:2]], axis=1)
    Z = jnp.concatenate(
        [jnp.transpose(gz.reshape(B, NP, 32), (0, 2, 1)), qt[:, 2:3]], axis=1)
    return _rri_features(X, Y, Z, B, NP)


# speedup vs baseline: 144.5778x; 1.0307x over previous
import functools

import jax
import jax.numpy as jnp
import numpy as np
from jax import lax
from jax.experimental import pallas as pl
from jax.experimental.pallas import tpu as pltpu
from jax.experimental.pallas import tpu_sc as plsc

RADIUS = 0.2
NSAMPLE = 32
QB = 128


def _cross(ax, ay, az, bx, by, bz):
    return (ay * bz - az * by, az * bx - ax * bz, ax * by - ay * bx)


def _bitonic_sort_tiles(vals):
    n = len(vals)
    k = 2
    while k <= n:
        j = k // 2
        while j >= 1:
            for i in range(n):
                l = i ^ j
                if l > i:
                    up = (i & k) == 0
                    a, b = vals[i], vals[l]
                    lo = jnp.minimum(a, b)
                    hi = jnp.maximum(a, b)
                    vals[i], vals[l] = (lo, hi) if up else (hi, lo)
            j //= 2
        k *= 2
    return vals


def _rri_kernel(xr, yr, zr, outr):
    px = xr[0, :32, :]
    py = yr[0, :32, :]
    pz = zr[0, :32, :]
    cx = xr[0, 32:33, :]
    cy = yr[0, 32:33, :]
    cz = zr[0, 32:33, :]

    pxi = px[:, None, :]
    pyi = py[:, None, :]
    pzi = pz[:, None, :]
    dx = pxi - px[None, :, :]
    dy = pyi - py[None, :, :]
    dz = pzi - pz[None, :, :]
    dis = jnp.sqrt((dx * dx + dy * dy) + dz * dz)

    mean_dis = jnp.sum(dis, axis=1) * jnp.float32(1.0 / 32.0)
    mx = jnp.max(mean_dis, axis=0, keepdims=True)
    ii = jax.lax.broadcasted_iota(jnp.int32, (32, QB), 0)
    tip = jnp.min(jnp.where(mean_dis == mx, ii, jnp.int32(32)), axis=0,
                  keepdims=True)
    sel = ii == tip
    zero = jnp.zeros((32, QB), jnp.float32)
    tx = jnp.sum(jnp.where(sel, px, zero), axis=0, keepdims=True)
    ty = jnp.sum(jnp.where(sel, py, zero), axis=0, keepdims=True)
    tz = jnp.sum(jnp.where(sel, pz, zero), axis=0, keepdims=True)

    ux, uy, uz = _cross(cx, cy, cz, px, py, pz)
    gx, gy, gz = _cross(ux, uy, uz, cx, cy, cz)
    gn = jnp.sqrt((gx * gx + gy * gy) + gz * gz)
    gx, gy, gz = gx / gn, gy / gn, gz / gn

    vx, vy, vz = _cross(cx, cy, cz, tx, ty, tz)
    tpx, tpy, tpz = _cross(vx, vy, vz, cx, cy, cz)
    tn = jnp.sqrt((tpx * tpx + tpy * tpy) + tpz * tpz)
    tpx, tpy, tpz = tpx / tn, tpy / tn, tpz / tn

    wx, wy, wz = _cross(gx, gy, gz, tpx, tpy, tpz)
    cr = jnp.sqrt((cx * cx + cy * cy) + cz * cz)
    inv = 1.0 / (cr + jnp.float32(1e-8))
    nnx, nny, nnz = cx * inv, cy * inv, cz * inv
    gp_sin = (wx * nnx + wy * nny) + wz * nnz

    tiles = [dis[i] for i in range(32)]
    tiles = _bitonic_sort_tiles(tiles)
    rows = [t * gp_sin for t in tiles]

    rows.append(jnp.sqrt((px * px + py * py) + pz * pz))
    outr[0] = jnp.stack(rows, axis=0)


def _rri_features(gxT, gyT, gzT, B, NP):
    out = pl.pallas_call(
        _rri_kernel,
        out_shape=jax.ShapeDtypeStruct((B, 33, 32, NP), jnp.float32),
        grid=(B, NP // QB),
        in_specs=[
            pl.BlockSpec((1, 33, QB), lambda b, q: (b, 0, q)),
            pl.BlockSpec((1, 33, QB), lambda b, q: (b, 0, q)),
            pl.BlockSpec((1, 33, QB), lambda b, q: (b, 0, q)),
        ],
        out_specs=pl.BlockSpec((1, 33, 32, QB), lambda b, q: (b, 0, 0, q)),
    )(gxT, gyT, gzT)
    return jnp.transpose(out, (0, 1, 3, 2))


def _ball_group_sc(xs, ys, zs, xx, qx, qy, qz, nn, B, N, NP):
    info = plsc.get_sparse_core_info()
    NC, NS = info.num_cores, info.num_subcores
    NW = NC * NS
    TQ = (B * NP) // NW
    TPB = NW // B
    r2 = np.float32(RADIUS * RADIUS)
    nchunks = N // 16

    mesh = plsc.VectorSubcoreMesh(core_axis_name="c", subcore_axis_name="s")

    @functools.partial(
        pl.kernel,
        mesh=mesh,
        compiler_params=pltpu.CompilerParams(needs_layout_passes=False),
        out_type=[jax.ShapeDtypeStruct((B * NP * 32,), jnp.float32)] * 3,
        scratch_types=[
            pltpu.VMEM((N,), jnp.float32),
            pltpu.VMEM((N,), jnp.float32),
            pltpu.VMEM((N,), jnp.float32),
            pltpu.VMEM((N,), jnp.float32),
            pltpu.VMEM((TQ + 16,), jnp.float32),
            pltpu.VMEM((TQ + 16,), jnp.float32),
            pltpu.VMEM((TQ + 16,), jnp.float32),
            pltpu.VMEM((TQ + 16,), jnp.float32),
            pltpu.VMEM((8 * 64,), jnp.int32),
            pltpu.VMEM((TQ * 32,), jnp.float32),
            pltpu.VMEM((TQ * 32,), jnp.float32),
            pltpu.VMEM((TQ * 32,), jnp.float32),
        ],
    )
    def k(xs_h, ys_h, zs_h, xx_h, qx_h, qy_h, qz_h, nn_h,
          gx_h, gy_h, gz_h,
          xs_v, ys_v, zs_v, xx_v, qx_v, qy_v, qz_v, nn_v, buf, ox, oy, oz):
        wid = lax.axis_index("s") * NC + lax.axis_index("c")
        qbase = wid * TQ
        pbase = (wid // TPB) * N
        pltpu.sync_copy(xs_h.at[pl.ds(pbase, N)], xs_v)
        pltpu.sync_copy(ys_h.at[pl.ds(pbase, N)], ys_v)
        pltpu.sync_copy(zs_h.at[pl.ds(pbase, N)], zs_v)
        pltpu.sync_copy(xx_h.at[pl.ds(pbase, N)], xx_v)
        pltpu.sync_copy(qx_h.at[pl.ds(qbase, TQ)], qx_v.at[pl.ds(0, TQ)])
        pltpu.sync_copy(qy_h.at[pl.ds(qbase, TQ)], qy_v.at[pl.ds(0, TQ)])
        pltpu.sync_copy(qz_h.at[pl.ds(qbase, TQ)], qz_v.at[pl.ds(0, TQ)])
        pltpu.sync_copy(nn_h.at[pl.ds(qbase, TQ)], nn_v.at[pl.ds(0, TQ)])

        lane = lax.iota(jnp.int32, 16)

        def bf16r(v):
            t = lax.bitcast_convert_type(v, jnp.int32)
            r = (t + jnp.int32(0x7FFF) + ((t >> 16) & 1)) & jnp.int32(-65536)
            return lax.bitcast_convert_type(r, jnp.float32)

        QG = 8

        def per_block(blk, carry):
            qb0 = blk * QG
            qxv = qx_v[pl.ds(qb0, 16)]
            qyv = qy_v[pl.ds(qb0, 16)]
            qzv = qz_v[pl.ds(qb0, 16)]
            nnv = nn_v[pl.ds(qb0, 16)]
            qxs = [qxv[j] for j in range(QG)]
            qys = [qyv[j] for j in range(QG)]
            qzs = [qzv[j] for j in range(QG)]
            nns = [nnv[j] for j in range(QG)]

            def cond(st):
                i = st[0]
                offs = st[1:]
                active = offs[0] < 32
                for o in offs[1:]:
                    active = active | (o < 32)
                return active & (i < nchunks // 2)

            def body(st):
                i = st[0]
                offs = list(st[1:])
                for u in range(2):
                    base = i * 32 + u * 16
                    px = bf16r(xs_v[pl.ds(base, 16)])
                    py = bf16r(ys_v[pl.ds(base, 16)])
                    pz = bf16r(zs_v[pl.ds(base, 16)])
                    xxv = xx_v[pl.ds(base, 16)]
                    ids = lane + base
                    for j in range(QG):
                        dot = (qxs[j] * px + qys[j] * py) + qzs[j] * pz
                        d2 = (nns[j] + xxv) - jnp.float32(2.0) * dot
                        m = d2 < r2
                        c = plsc.all_reduce_population_count(m)[0]
                        o = offs[j]
                        plsc.store_compressed(buf.at[pl.ds(j * 64 + o, 16)],
                                              ids, mask=m)
                        offs[j] = jnp.where(o < 32, o + c, o)
                return (i + 1, *offs)

            st = lax.while_loop(cond, body, (jnp.int32(0),) + (jnp.int32(0),) * QG)
            offs = st[1:]

            for j in range(QG):
                off = offs[j]
                jb = j * 64
                i0 = buf[pl.ds(jb, 16)]
                i1 = buf[pl.ds(jb + 16, 16)]
                first = jnp.where(off > 0, i0[0], jnp.int32(0))
                i0 = jnp.where(lane < off, i0, first)
                i1 = jnp.where(lane + 16 < off, i1, first)
                qo = (qb0 + j) * 32
                ox[pl.ds(qo, 16)] = plsc.load_gather(xs_v, [i0])
                ox[pl.ds(qo + 16, 16)] = plsc.load_gather(xs_v, [i1])
                oy[pl.ds(qo, 16)] = plsc.load_gather(ys_v, [i0])
                oy[pl.ds(qo + 16, 16)] = plsc.load_gather(ys_v, [i1])
                oz[pl.ds(qo, 16)] = plsc.load_gather(zs_v, [i0])
                oz[pl.ds(qo + 16, 16)] = plsc.load_gather(zs_v, [i1])
            return carry

        lax.fori_loop(0, TQ // QG, per_block, 0)
        pltpu.sync_copy(ox, gx_h.at[pl.ds(qbase * 32, TQ * 32)])
        pltpu.sync_copy(oy, gy_h.at[pl.ds(qbase * 32, TQ * 32)])
        pltpu.sync_copy(oz, gz_h.at[pl.ds(qbase * 32, TQ * 32)])

    return k(xs, ys, zs, xx, qx, qy, qz, nn)


def kernel(xyz, new_xyz):
    B, N, _ = xyz.shape
    NP = new_xyz.shape[1]
    xt = jnp.transpose(xyz, (0, 2, 1))
    qt = jnp.transpose(new_xyz, (0, 2, 1))
    xx = jnp.sum(xyz * xyz, axis=-1)
    nn = jnp.sum(new_xyz * new_xyz, axis=-1)
    qtb = lax.bitcast_convert_type(qt, jnp.int32)
    qtb = (qtb + jnp.int32(0x7FFF) + ((qtb >> 16) & 1)) & jnp.int32(-65536)
    qb = lax.bitcast_convert_type(qtb, jnp.float32)
    gx, gy, gz = _ball_group_sc(
        xt[:, 0].reshape(-1), xt[:, 1].reshape(-1), xt[:, 2].reshape(-1),
        xx.reshape(-1),
        qb[:, 0].reshape(-1), qb[:, 1].reshape(-1), qb[:, 2].reshape(-1),
        nn.reshape(-1), B, N, NP)
    X = jnp.concatenate(
        [jnp.transpose(gx.reshape(B, NP, 32), (0, 2, 1)), qt[:, 0:1]], axis=1)
    Y = jnp.concatenate(
        [jnp.transpose(gy.reshape(B, NP, 32), (0, 2, 1)), qt[:, 1:2]], axis=1)
    Z = jnp.concatenate(
        [jnp.transpose(gz.reshape(B, NP, 32), (0, 2, 1)), qt[:, 2:3]], axis=1)
    return _rri_features(X, Y, Z, B, NP)


# speedup vs baseline: 155.1863x; 1.0734x over previous
import functools

import jax
import jax.numpy as jnp
import numpy as np
from jax import lax
from jax.experimental import pallas as pl
from jax.experimental.pallas import tpu as pltpu
from jax.experimental.pallas import tpu_sc as plsc

RADIUS = 0.2
NSAMPLE = 32
QB = 128


def _cross(ax, ay, az, bx, by, bz):
    return (ay * bz - az * by, az * bx - ax * bz, ax * by - ay * bx)


def _bitonic_sort_tiles(vals):
    n = len(vals)
    k = 2
    while k <= n:
        j = k // 2
        while j >= 1:
            for i in range(n):
                l = i ^ j
                if l > i:
                    up = (i & k) == 0
                    a, b = vals[i], vals[l]
                    lo = jnp.minimum(a, b)
                    hi = jnp.maximum(a, b)
                    vals[i], vals[l] = (lo, hi) if up else (hi, lo)
            j //= 2
        k *= 2
    return vals


def _rri_kernel(xr, yr, zr, outr):
    px = xr[0, :32, :]
    py = yr[0, :32, :]
    pz = zr[0, :32, :]
    cx = xr[0, 32:33, :]
    cy = yr[0, 32:33, :]
    cz = zr[0, 32:33, :]

    pxi = px[:, None, :]
    pyi = py[:, None, :]
    pzi = pz[:, None, :]
    dx = pxi - px[None, :, :]
    dy = pyi - py[None, :, :]
    dz = pzi - pz[None, :, :]
    dis = jnp.sqrt((dx * dx + dy * dy) + dz * dz)

    mean_dis = jnp.sum(dis, axis=1) * jnp.float32(1.0 / 32.0)
    mx = jnp.max(mean_dis, axis=0, keepdims=True)
    ii = jax.lax.broadcasted_iota(jnp.int32, (32, QB), 0)
    tip = jnp.min(jnp.where(mean_dis == mx, ii, jnp.int32(32)), axis=0,
                  keepdims=True)
    sel = ii == tip
    zero = jnp.zeros((32, QB), jnp.float32)
    tx = jnp.sum(jnp.where(sel, px, zero), axis=0, keepdims=True)
    ty = jnp.sum(jnp.where(sel, py, zero), axis=0, keepdims=True)
    tz = jnp.sum(jnp.where(sel, pz, zero), axis=0, keepdims=True)

    ux, uy, uz = _cross(cx, cy, cz, px, py, pz)
    gx, gy, gz = _cross(ux, uy, uz, cx, cy, cz)
    gn = jnp.sqrt((gx * gx + gy * gy) + gz * gz)
    gx, gy, gz = gx / gn, gy / gn, gz / gn

    vx, vy, vz = _cross(cx, cy, cz, tx, ty, tz)
    tpx, tpy, tpz = _cross(vx, vy, vz, cx, cy, cz)
    tn = jnp.sqrt((tpx * tpx + tpy * tpy) + tpz * tpz)
    tpx, tpy, tpz = tpx / tn, tpy / tn, tpz / tn

    wx, wy, wz = _cross(gx, gy, gz, tpx, tpy, tpz)
    cr = jnp.sqrt((cx * cx + cy * cy) + cz * cz)
    inv = 1.0 / (cr + jnp.float32(1e-8))
    nnx, nny, nnz = cx * inv, cy * inv, cz * inv
    gp_sin = (wx * nnx + wy * nny) + wz * nnz

    tiles = [dis[i] for i in range(32)]
    tiles = _bitonic_sort_tiles(tiles)
    rows = [t * gp_sin for t in tiles]

    rows.append(jnp.sqrt((px * px + py * py) + pz * pz))
    outr[0] = jnp.stack(rows, axis=0)


def _rri_features(gxT, gyT, gzT, B, NP):
    out = pl.pallas_call(
        _rri_kernel,
        out_shape=jax.ShapeDtypeStruct((B, 33, 32, NP), jnp.float32),
        grid=(B, NP // QB),
        in_specs=[
            pl.BlockSpec((1, 33, QB), lambda b, q: (b, 0, q)),
            pl.BlockSpec((1, 33, QB), lambda b, q: (b, 0, q)),
            pl.BlockSpec((1, 33, QB), lambda b, q: (b, 0, q)),
        ],
        out_specs=pl.BlockSpec((1, 33, 32, QB), lambda b, q: (b, 0, 0, q)),
    )(gxT, gyT, gzT)
    return jnp.transpose(out, (0, 1, 3, 2))


def _ball_group_sc(xs, ys, zs, xx, qx, qy, qz, nn, B, N, NP):
    info = plsc.get_sparse_core_info()
    NC, NS = info.num_cores, info.num_subcores
    NW = NC * NS
    TQ = (B * NP) // NW
    TPB = NW // B
    r2 = np.float32(RADIUS * RADIUS)
    nchunks = N // 16

    mesh = plsc.VectorSubcoreMesh(core_axis_name="c", subcore_axis_name="s")

    @functools.partial(
        pl.kernel,
        mesh=mesh,
        compiler_params=pltpu.CompilerParams(needs_layout_passes=False),
        out_type=[jax.ShapeDtypeStruct((B * NP * 32,), jnp.float32)] * 3,
        scratch_types=[
            pltpu.VMEM((N,), jnp.float32),
            pltpu.VMEM((N,), jnp.float32),
            pltpu.VMEM((N,), jnp.float32),
            pltpu.VMEM((N,), jnp.float32),
            pltpu.VMEM((TQ + 16,), jnp.float32),
            pltpu.VMEM((TQ + 16,), jnp.float32),
            pltpu.VMEM((TQ + 16,), jnp.float32),
            pltpu.VMEM((TQ + 16,), jnp.float32),
            pltpu.VMEM((8 * 64,), jnp.int32),
            pltpu.VMEM((TQ * 32,), jnp.float32),
            pltpu.VMEM((TQ * 32,), jnp.float32),
            pltpu.VMEM((TQ * 32,), jnp.float32),
        ],
    )
    def k(xs_h, ys_h, zs_h, xx_h, qx_h, qy_h, qz_h, nn_h,
          gx_h, gy_h, gz_h,
          xs_v, ys_v, zs_v, xx_v, qx_v, qy_v, qz_v, nn_v, buf, ox, oy, oz):
        wid = lax.axis_index("s") * NC + lax.axis_index("c")
        qbase = wid * TQ
        pbase = (wid // TPB) * N
        pltpu.sync_copy(xs_h.at[pl.ds(pbase, N)], xs_v)
        pltpu.sync_copy(ys_h.at[pl.ds(pbase, N)], ys_v)
        pltpu.sync_copy(zs_h.at[pl.ds(pbase, N)], zs_v)
        pltpu.sync_copy(xx_h.at[pl.ds(pbase, N)], xx_v)
        pltpu.sync_copy(qx_h.at[pl.ds(qbase, TQ)], qx_v.at[pl.ds(0, TQ)])
        pltpu.sync_copy(qy_h.at[pl.ds(qbase, TQ)], qy_v.at[pl.ds(0, TQ)])
        pltpu.sync_copy(qz_h.at[pl.ds(qbase, TQ)], qz_v.at[pl.ds(0, TQ)])
        pltpu.sync_copy(nn_h.at[pl.ds(qbase, TQ)], nn_v.at[pl.ds(0, TQ)])

        lane = lax.iota(jnp.int32, 16)

        def bf16r(v):
            t = lax.bitcast_convert_type(v, jnp.int32)
            r = (t + jnp.int32(0x7FFF) + ((t >> 16) & 1)) & jnp.int32(-65536)
            return lax.bitcast_convert_type(r, jnp.float32)

        QG = 8

        def per_block(blk, carry):
            qb0 = blk * QG
            qxv = qx_v[pl.ds(qb0, 16)]
            qyv = qy_v[pl.ds(qb0, 16)]
            qzv = qz_v[pl.ds(qb0, 16)]
            nnv = nn_v[pl.ds(qb0, 16)]
            qxs = [qxv[j] for j in range(QG)]
            qys = [qyv[j] for j in range(QG)]
            qzs = [qzv[j] for j in range(QG)]
            nns = [nnv[j] for j in range(QG)]

            def cond(st):
                i, ov = st
                act = plsc.all_reduce_population_count(ov < 32)[0]
                return (act > 0) & (i < nchunks)

            def body(st):
                i, ov = st
                offs = [ov[j] for j in range(QG)]
                base = i * 16
                px = bf16r(xs_v[pl.ds(base, 16)])
                py = bf16r(ys_v[pl.ds(base, 16)])
                pz = bf16r(zs_v[pl.ds(base, 16)])
                xxv = xx_v[pl.ds(base, 16)]
                ids = lane + base
                cnt = jnp.zeros((16,), jnp.int32)
                for j in range(QG):
                    dot = (qxs[j] * px + qys[j] * py) + qzs[j] * pz
                    d2 = (nns[j] + xxv) - jnp.float32(2.0) * dot
                    m = d2 < r2
                    pc = plsc.all_reduce_population_count(m)
                    cnt = jnp.where(lane == j, pc, cnt)
                    plsc.store_compressed(
                        buf.at[pl.ds(j * 64 + offs[j], 16)], ids, mask=m)
                ov = jnp.where(ov < 32, ov + cnt, ov)
                return i + 1, ov

            ov0 = jnp.where(lane < QG, jnp.int32(0), jnp.int32(32))
            _, ovf = lax.while_loop(cond, body, (jnp.int32(0), ov0))

            for j in range(QG):
                off = ovf[j]
                jb = j * 64
                i0 = buf[pl.ds(jb, 16)]
                i1 = buf[pl.ds(jb + 16, 16)]
                first = jnp.where(off > 0, i0[0], jnp.int32(0))
                i0 = jnp.where(lane < off, i0, first)
                i1 = jnp.where(lane + 16 < off, i1, first)
                qo = (qb0 + j) * 32
                ox[pl.ds(qo, 16)] = plsc.load_gather(xs_v, [i0])
                ox[pl.ds(qo + 16, 16)] = plsc.load_gather(xs_v, [i1])
                oy[pl.ds(qo, 16)] = plsc.load_gather(ys_v, [i0])
                oy[pl.ds(qo + 16, 16)] = plsc.load_gather(ys_v, [i1])
                oz[pl.ds(qo, 16)] = plsc.load_gather(zs_v, [i0])
                oz[pl.ds(qo + 16, 16)] = plsc.load_gather(zs_v, [i1])
            return carry

        lax.fori_loop(0, TQ // QG, per_block, 0)
        pltpu.sync_copy(ox, gx_h.at[pl.ds(qbase * 32, TQ * 32)])
        pltpu.sync_copy(oy, gy_h.at[pl.ds(qbase * 32, TQ * 32)])
        pltpu.sync_copy(oz, gz_h.at[pl.ds(qbase * 32, TQ * 32)])

    return k(xs, ys, zs, xx, qx, qy, qz, nn)


def kernel(xyz, new_xyz):
    B, N, _ = xyz.shape
    NP = new_xyz.shape[1]
    xt = jnp.transpose(xyz, (0, 2, 1))
    qt = jnp.transpose(new_xyz, (0, 2, 1))
    xx = jnp.sum(xyz * xyz, axis=-1)
    nn = jnp.sum(new_xyz * new_xyz, axis=-1)
    qtb = lax.bitcast_convert_type(qt, jnp.int32)
    qtb = (qtb + jnp.int32(0x7FFF) + ((qtb >> 16) & 1)) & jnp.int32(-65536)
    qb = lax.bitcast_convert_type(qtb, jnp.float32)
    gx, gy, gz = _ball_group_sc(
        xt[:, 0].reshape(-1), xt[:, 1].reshape(-1), xt[:, 2].reshape(-1),
        xx.reshape(-1),
        qb[:, 0].reshape(-1), qb[:, 1].reshape(-1), qb[:, 2].reshape(-1),
        nn.reshape(-1), B, N, NP)
    X = jnp.concatenate(
        [jnp.transpose(gx.reshape(B, NP, 32), (0, 2, 1)), qt[:, 0:1]], axis=1)
    Y = jnp.concatenate(
        [jnp.transpose(gy.reshape(B, NP, 32), (0, 2, 1)), qt[:, 1:2]], axis=1)
    Z = jnp.concatenate(
        [jnp.transpose(gz.reshape(B, NP, 32), (0, 2, 1)), qt[:, 2:3]], axis=1)
    return _rri_features(X, Y, Z, B, NP)
